# Initial kernel scaffold; baseline (speedup 1.0000x reference)
import jax
import jax.numpy as jnp
from jax.experimental import pallas as pl


def kernel(support_features, query_features, support_labels, query_labels):
    raise NotImplementedError("write your pallas kernel here")


# speedup vs baseline: 9.9431x; 9.9431x over previous
import jax
import jax.numpy as jnp
from jax.experimental import pallas as pl
from jax.experimental.pallas import tpu as pltpu

_S = 16384
_Q = 4096
_D = 128
_K = 5

_BQ = 256
_CH = 2048


def _ncm_body(scls_ref, qcls_ref, q_ref, s_ref, out_ref, scores_ref, ssq_ref):
    i = pl.program_id(0)

    @pl.when(i == 0)
    def _compute_ssq():
        s = s_ref[...]
        ssq_ref[...] = jax.lax.dot_general(
            jnp.ones((1, _D), jnp.float32), s * s,
            (((1,), (1,)), ((), ())),
            preferred_element_type=jnp.float32)

    q = q_ref[...]

    for c in range(_S // _CH):
        sl = pl.ds(c * _CH, _CH)
        dots = jax.lax.dot_general(
            q, s_ref[sl, :], (((1,), (1,)), ((), ())),
            preferred_element_type=jnp.float32)
        scores_ref[:, sl] = ssq_ref[:, sl] - 2.0 * dots

    qc = qcls_ref[0, 0, :][:, None]

    m = jnp.full((_BQ, 1), jnp.inf, jnp.float32)
    idx_m = jnp.full((_BQ, 1), _S, jnp.int32)
    for c in range(_S // _CH):
        sl = pl.ds(c * _CH, _CH)
        sc = scores_ref[:, sl]
        mask = scls_ref[:, sl] == qc
        msc = jnp.where(mask, sc, jnp.inf)
        cm = jnp.min(msc, axis=1, keepdims=True)
        iota = jax.lax.broadcasted_iota(jnp.int32, (_BQ, _CH), 1) + c * _CH
        cidx = jnp.min(jnp.where(mask & (msc == cm), iota, _S),
                       axis=1, keepdims=True)
        better = cm < m
        same = cm == m
        idx_m = jnp.where(better, cidx,
                          jnp.where(same, jnp.minimum(idx_m, cidx), idx_m))
        m = jnp.minimum(m, cm)

    cnt = jnp.zeros((_BQ, 1), jnp.int32)
    for c in range(_S // _CH):
        sl = pl.ds(c * _CH, _CH)
        sc = scores_ref[:, sl]
        iota = jax.lax.broadcasted_iota(jnp.int32, (_BQ, _CH), 1) + c * _CH
        ahead = (sc < m) | ((sc == m) & (iota < idx_m))
        cnt += jnp.sum(ahead.astype(jnp.int32), axis=1, keepdims=True)

    part = jnp.sum((cnt < _K).astype(jnp.float32)) * (1.0 / _Q)

    @pl.when(i == 0)
    def _init_out():
        out_ref[...] = jnp.zeros((1, 1), jnp.float32)

    out_ref[...] = out_ref[...] + part


def kernel(support_features, query_features, support_labels, query_labels):
    scls = support_labels[:, 0].reshape(1, _S)
    qcls = query_labels[:, 0].reshape(_Q // _BQ, 1, _BQ)

    acc = pl.pallas_call(
        _ncm_body,
        grid=(_Q // _BQ,),
        in_specs=[
            pl.BlockSpec((1, _S), lambda i: (0, 0)),
            pl.BlockSpec((1, 1, _BQ), lambda i: (i, 0, 0)),
            pl.BlockSpec((_BQ, _D), lambda i: (i, 0)),
            pl.BlockSpec((_S, _D), lambda i: (0, 0)),
        ],
        out_specs=pl.BlockSpec((1, 1), lambda i: (0, 0)),
        out_shape=jax.ShapeDtypeStruct((1, 1), jnp.float32),
        scratch_shapes=[
            pltpu.VMEM((_BQ, _S), jnp.float32),
            pltpu.VMEM((1, _S), jnp.float32),
        ],
        compiler_params=pltpu.CompilerParams(
            dimension_semantics=("arbitrary",),
            vmem_limit_bytes=60 * 1024 * 1024,
        ),
    )(scls, qcls, query_features, support_features)
    return acc[0, 0]


# speedup vs baseline: 16.8638x; 1.6960x over previous
import jax
import jax.numpy as jnp
from jax.experimental import pallas as pl
from jax.experimental.pallas import tpu as pltpu

_S = 16384
_Q = 4096
_D = 128
_K = 5

_BQ = 256
_CH = 2048


def _ncm_body(scls_ref, qcls_ref, q_ref, s_ref, out_ref,
              scores_ref, ssq_ref, adj_ref):
    i = pl.program_id(0)

    @pl.when(i == 0)
    def _compute_ssq():
        s = s_ref[...]
        ssq_ref[...] = jax.lax.dot_general(
            jnp.ones((1, _D), jnp.float32), s * s,
            (((1,), (1,)), ((), ())),
            preferred_element_type=jnp.float32)

    q = q_ref[...]
    qc = qcls_ref[0, 0, :][:, None]

    m = jnp.full((_BQ, 1), jnp.inf, jnp.float32)
    for c in range(_S // _CH):
        sl = pl.ds(c * _CH, _CH)
        dots = jax.lax.dot_general(
            q, s_ref[sl, :], (((1,), (1,)), ((), ())),
            preferred_element_type=jnp.float32)
        sc = ssq_ref[:, sl] - 2.0 * dots
        scores_ref[:, sl] = sc
        msc = jnp.where(scls_ref[:, sl] == qc, sc, jnp.inf)
        m = jnp.minimum(m, jnp.min(msc, axis=1, keepdims=True))

    cnt_lt = jnp.zeros((_BQ, 1), jnp.int32)
    cnt_eq = jnp.zeros((_BQ, 1), jnp.int32)
    for c in range(_S // _CH):
        sl = pl.ds(c * _CH, _CH)
        sc = scores_ref[:, sl]
        cnt_lt += jnp.sum((sc < m).astype(jnp.int32), axis=1, keepdims=True)
        cnt_eq += jnp.sum((sc == m).astype(jnp.int32), axis=1, keepdims=True)

    amb = (cnt_lt < _K) & (cnt_lt + cnt_eq > _K)
    adj_ref[...] = jnp.zeros((_BQ, 1), jnp.int32)

    @pl.when(jnp.any(amb))
    def _resolve_ties():
        idx_m = jnp.full((_BQ, 1), _S, jnp.int32)
        for c in range(_S // _CH):
            sl = pl.ds(c * _CH, _CH)
            sc = scores_ref[:, sl]
            hit = (sc == m) & (scls_ref[:, sl] == qc)
            iota = jax.lax.broadcasted_iota(jnp.int32, (_BQ, _CH), 1) + c * _CH
            idx_m = jnp.minimum(
                idx_m, jnp.min(jnp.where(hit, iota, _S), axis=1, keepdims=True))
        eq_before = jnp.zeros((_BQ, 1), jnp.int32)
        for c in range(_S // _CH):
            sl = pl.ds(c * _CH, _CH)
            sc = scores_ref[:, sl]
            iota = jax.lax.broadcasted_iota(jnp.int32, (_BQ, _CH), 1) + c * _CH
            eq_before += jnp.sum(((sc == m) & (iota < idx_m)).astype(jnp.int32),
                                 axis=1, keepdims=True)
        adj_ref[...] = eq_before

    cnt = cnt_lt + adj_ref[...]
    part = jnp.sum((cnt < _K).astype(jnp.float32)) * (1.0 / _Q)

    @pl.when(i == 0)
    def _init_out():
        out_ref[...] = jnp.zeros((1, 1), jnp.float32)

    out_ref[...] = out_ref[...] + part


def kernel(support_features, query_features, support_labels, query_labels):
    scls = support_labels[:, 0].reshape(1, _S)
    qcls = query_labels[:, 0].reshape(_Q // _BQ, 1, _BQ)

    acc = pl.pallas_call(
        _ncm_body,
        grid=(_Q // _BQ,),
        in_specs=[
            pl.BlockSpec((1, _S), lambda i: (0, 0)),
            pl.BlockSpec((1, 1, _BQ), lambda i: (i, 0, 0)),
            pl.BlockSpec((_BQ, _D), lambda i: (i, 0)),
            pl.BlockSpec((_S, _D), lambda i: (0, 0)),
        ],
        out_specs=pl.BlockSpec((1, 1), lambda i: (0, 0)),
        out_shape=jax.ShapeDtypeStruct((1, 1), jnp.float32),
        scratch_shapes=[
            pltpu.VMEM((_BQ, _S), jnp.float32),
            pltpu.VMEM((1, _S), jnp.float32),
            pltpu.VMEM((_BQ, 1), jnp.int32),
        ],
        compiler_params=pltpu.CompilerParams(
            dimension_semantics=("arbitrary",),
            vmem_limit_bytes=60 * 1024 * 1024,
        ),
    )(scls, qcls, query_features, support_features)
    return acc[0, 0]
